# baseline (speedup 1.0000x reference)
import functools

import jax
import jax.numpy as jnp
from jax import lax
from jax.experimental import pallas as pl
from jax.experimental.pallas import tpu as pltpu
from jax.experimental.pallas import tpu_sc as plsc

VOCAB = 1000
DIM = 8
HIDDEN = 128
_TAB = 2 * VOCAB * DIM

_NC = 2
_NS = 16
_NW = _NC * _NS


def _make_gather(batch: int):
    rows_per_w = batch // _NW
    groups = rows_per_w // 16
    prows_w = rows_per_w // 8
    mesh = plsc.VectorSubcoreMesh(core_axis_name="c", subcore_axis_name="s")

    @functools.partial(
        pl.kernel,
        out_type=jax.ShapeDtypeStruct((batch // 8, 16 * DIM), jnp.float32),
        mesh=mesh,
        scratch_types=[
            pltpu.VMEM((_TAB,), jnp.float32),
            pltpu.VMEM((rows_per_w,), jnp.int32),
            pltpu.VMEM((rows_per_w,), jnp.int32),
            pltpu.VMEM((prows_w, 16 * DIM), jnp.float32),
            pltpu.SemaphoreType.DMA,
            pltpu.SemaphoreType.DMA,
            pltpu.SemaphoreType.DMA,
        ],
        compiler_params=pltpu.CompilerParams(needs_layout_passes=False),
    )
    def gather_kernel(tab_hbm, feat_hbm, out_hbm, tab_v, cu_v, co_v, obuf,
                      s0, s1, s2):
        wid = lax.axis_index("s") * _NC + lax.axis_index("c")
        base = wid * rows_per_w
        c0 = pltpu.async_copy(tab_hbm, tab_v, s0)
        c1 = pltpu.async_copy(feat_hbm.at[0, pl.ds(base, rows_per_w)], cu_v,
                              s1)
        c2 = pltpu.async_copy(feat_hbm.at[1, pl.ds(base, rows_per_w)], co_v,
                              s2)
        c1.wait()
        c2.wait()
        c0.wait()

        lane = lax.iota(jnp.int32, 16)
        is_cust = lane < DIM

        def body(g, carry):
            cuv = cu_v[pl.ds(g * 16, 16)] * DIM
            cov = co_v[pl.ds(g * 16, 16)] * DIM + (VOCAB * DIM - DIM)
            for t in range(16):
                addr = lane + jnp.where(is_cust, cuv[t], cov[t])
                v = plsc.load_gather(tab_v, [addr])
                obuf[2 * g + t // 8, pl.ds((t % 8) * 16, 16)] = v
            return carry

        @plsc.parallel_loop(0, groups)
        def _(g):
            body(g, 0)

        pltpu.sync_copy(obuf, out_hbm.at[pl.ds(wid * prows_w, prows_w), :])

    return gather_kernel


def _mlp_body(p_ref, w1_ref, b1_ref, w2_ref, b2_ref, out_ref):
    f32, bf16 = jnp.float32, jnp.bfloat16
    w1t = jnp.concatenate([jnp.concatenate([w1_ref[...]] * 8, axis=1)] * 8,
                          axis=0)
    r1 = lax.broadcasted_iota(jnp.int32, (128, 8 * HIDDEN), 0) // 16
    c1 = lax.broadcasted_iota(jnp.int32, (128, 8 * HIDDEN), 1) // HIDDEN
    bd1 = jnp.where(r1 == c1, w1t, 0.0).astype(bf16)
    b1t = jnp.concatenate([b1_ref[...]] * 8, axis=1)

    h = jnp.dot(p_ref[...].astype(bf16), bd1, preferred_element_type=f32)
    h = jnp.maximum(h + b1t, 0.0)

    w2t = jnp.concatenate([jnp.concatenate([w2_ref[...]] * 8, axis=0)] * 8,
                          axis=1)
    r2 = lax.broadcasted_iota(jnp.int32, (8 * HIDDEN, 8), 0) // HIDDEN
    c2 = lax.broadcasted_iota(jnp.int32, (8 * HIDDEN, 8), 1)
    bd2 = jnp.where(r2 == c2, w2t, 0.0).astype(bf16)

    z = jnp.dot(h.astype(bf16), bd2, preferred_element_type=f32)
    out_ref[...] = 1.0 / (1.0 + jnp.exp(-(z + b2_ref[...])))


def kernel(features, customers_emb, content_emb, W1, b1, W2, b2):
    batch = features.shape[0]
    prows = batch // 8

    tab = jnp.concatenate([customers_emb, content_emb], axis=0).reshape(-1)
    ft = features.T
    packed = _make_gather(batch)(tab, ft)

    zp = pl.pallas_call(
        _mlp_body,
        grid=(4,),
        in_specs=[
            pl.BlockSpec((prows // 4, 16 * DIM), lambda i: (i, 0)),
            pl.BlockSpec((2 * DIM, HIDDEN), lambda i: (0, 0)),
            pl.BlockSpec((1, HIDDEN), lambda i: (0, 0)),
            pl.BlockSpec((HIDDEN, 1), lambda i: (0, 0)),
            pl.BlockSpec((1, 1), lambda i: (0, 0)),
        ],
        out_specs=pl.BlockSpec((prows // 4, 8), lambda i: (i, 0)),
        out_shape=jax.ShapeDtypeStruct((prows, 8), jnp.float32),
    )(packed, W1, b1.reshape(1, HIDDEN), W2, b2.reshape(1, 1))

    return zp.reshape(batch, 1)


# speedup vs baseline: 1.0036x; 1.0036x over previous
import functools

import jax
import jax.numpy as jnp
from jax import lax
from jax.experimental import pallas as pl
from jax.experimental.pallas import tpu as pltpu
from jax.experimental.pallas import tpu_sc as plsc

VOCAB = 1000
DIM = 8
HIDDEN = 128
_TAB = 2 * VOCAB * DIM

_NC = 2
_NS = 16
_NW = _NC * _NS


def _make_gather(batch: int):
    rows_per_w = batch // _NW
    groups = rows_per_w // 16
    prows_w = rows_per_w // 8
    mesh = plsc.VectorSubcoreMesh(core_axis_name="c", subcore_axis_name="s")

    @functools.partial(
        pl.kernel,
        out_type=jax.ShapeDtypeStruct((batch // 8, 16 * DIM), jnp.float32),
        mesh=mesh,
        scratch_types=[
            pltpu.VMEM((_TAB,), jnp.float32),
            pltpu.VMEM((rows_per_w,), jnp.int32),
            pltpu.VMEM((rows_per_w,), jnp.int32),
            pltpu.VMEM((prows_w, 16 * DIM), jnp.float32),
            pltpu.SemaphoreType.DMA,
            pltpu.SemaphoreType.DMA,
            pltpu.SemaphoreType.DMA,
        ],
        compiler_params=pltpu.CompilerParams(needs_layout_passes=False),
    )
    def gather_kernel(tab_hbm, feat_hbm, out_hbm, tab_v, cu_v, co_v, obuf,
                      s0, s1, s2):
        wid = lax.axis_index("s") * _NC + lax.axis_index("c")
        base = wid * rows_per_w
        c0 = pltpu.async_copy(tab_hbm, tab_v, s0)
        c1 = pltpu.async_copy(feat_hbm.at[0, pl.ds(base, rows_per_w)], cu_v,
                              s1)
        c2 = pltpu.async_copy(feat_hbm.at[1, pl.ds(base, rows_per_w)], co_v,
                              s2)
        c1.wait()
        c2.wait()
        c0.wait()

        lane = lax.iota(jnp.int32, 16)
        is_cust = lane < DIM

        def body(g, carry):
            cuv = cu_v[pl.ds(g * 16, 16)] * DIM
            cov = co_v[pl.ds(g * 16, 16)] * DIM + (VOCAB * DIM - DIM)
            for t in range(16):
                addr = lane + jnp.where(is_cust, cuv[t], cov[t])
                v = plsc.load_gather(tab_v, [addr])
                obuf[2 * g + t // 8, pl.ds((t % 8) * 16, 16)] = v
            return carry

        @plsc.parallel_loop(0, groups)
        def _(g):
            body(g, 0)

        pltpu.sync_copy(obuf, out_hbm.at[pl.ds(wid * prows_w, prows_w), :])

    return gather_kernel


def _mlp_body(p_ref, w1_ref, b1_ref, w2_ref, b2_ref, out_ref,
              bd1_s, bd2_s):
    f32, bf16 = jnp.float32, jnp.bfloat16

    @pl.when(pl.program_id(0) == 0)
    def _build():
        w1t = jnp.concatenate(
            [jnp.concatenate([w1_ref[...]] * 8, axis=1)] * 8, axis=0)
        r1 = lax.broadcasted_iota(jnp.int32, (128, 8 * HIDDEN), 0) // 16
        c1 = lax.broadcasted_iota(jnp.int32, (128, 8 * HIDDEN), 1) // HIDDEN
        bd1_s[...] = jnp.where(r1 == c1, w1t, 0.0).astype(bf16)
        w2t = jnp.concatenate(
            [jnp.concatenate([w2_ref[...]] * 8, axis=0)] * 8, axis=1)
        r2 = lax.broadcasted_iota(jnp.int32, (8 * HIDDEN, 8), 0) // HIDDEN
        c2 = lax.broadcasted_iota(jnp.int32, (8 * HIDDEN, 8), 1)
        bd2_s[...] = jnp.where(r2 == c2, w2t, 0.0).astype(bf16)

    b1t = jnp.concatenate([b1_ref[...]] * 8, axis=1)
    h = jnp.dot(p_ref[...].astype(bf16), bd1_s[...],
                preferred_element_type=f32)
    h = jnp.maximum(h + b1t, 0.0)
    z = jnp.dot(h.astype(bf16), bd2_s[...], preferred_element_type=f32)
    out_ref[...] = 1.0 / (1.0 + jnp.exp(-(z + b2_ref[...])))


def kernel(features, customers_emb, content_emb, W1, b1, W2, b2):
    batch = features.shape[0]
    prows = batch // 8

    tab = jnp.concatenate([customers_emb, content_emb], axis=0).reshape(-1)
    ft = features.T
    packed = _make_gather(batch)(tab, ft)

    zp = pl.pallas_call(
        _mlp_body,
        grid=(2,),
        in_specs=[
            pl.BlockSpec((prows // 2, 16 * DIM), lambda i: (i, 0)),
            pl.BlockSpec((2 * DIM, HIDDEN), lambda i: (0, 0)),
            pl.BlockSpec((1, HIDDEN), lambda i: (0, 0)),
            pl.BlockSpec((HIDDEN, 1), lambda i: (0, 0)),
            pl.BlockSpec((1, 1), lambda i: (0, 0)),
        ],
        out_specs=pl.BlockSpec((prows // 2, 8), lambda i: (i, 0)),
        out_shape=jax.ShapeDtypeStruct((prows, 8), jnp.float32),
        scratch_shapes=[
            pltpu.VMEM((128, 8 * HIDDEN), jnp.bfloat16),
            pltpu.VMEM((8 * HIDDEN, 8), jnp.bfloat16),
        ],
    )(packed, W1, b1.reshape(1, HIDDEN), W2, b2.reshape(1, 1))

    return zp.reshape(batch, 1)


# speedup vs baseline: 1.0104x; 1.0067x over previous
import functools

import jax
import jax.numpy as jnp
from jax import lax
from jax.experimental import pallas as pl
from jax.experimental.pallas import tpu as pltpu
from jax.experimental.pallas import tpu_sc as plsc

VOCAB = 1000
DIM = 8
HIDDEN = 128
_TAB = 2 * VOCAB * DIM

_NC = 2
_NS = 16
_NW = _NC * _NS


def _make_gather(batch: int):
    rows_per_w = batch // _NW
    groups = rows_per_w // 16
    prows_w = rows_per_w // 8
    mesh = plsc.VectorSubcoreMesh(core_axis_name="c", subcore_axis_name="s")

    @functools.partial(
        pl.kernel,
        out_type=jax.ShapeDtypeStruct((batch // 8, 16 * DIM), jnp.float32),
        mesh=mesh,
        scratch_types=[
            pltpu.VMEM((_TAB,), jnp.float32),
            pltpu.VMEM((rows_per_w,), jnp.int32),
            pltpu.VMEM((rows_per_w,), jnp.int32),
            pltpu.VMEM((prows_w, 16 * DIM), jnp.float32),
            pltpu.SemaphoreType.DMA,
            pltpu.SemaphoreType.DMA,
            pltpu.SemaphoreType.DMA,
        ],
        compiler_params=pltpu.CompilerParams(needs_layout_passes=False),
    )
    def gather_kernel(tab_hbm, feat_hbm, out_hbm, tab_v, cu_v, co_v, obuf,
                      s0, s1, s2):
        wid = lax.axis_index("s") * _NC + lax.axis_index("c")
        base = wid * rows_per_w
        c0 = pltpu.async_copy(tab_hbm, tab_v, s0)
        c1 = pltpu.async_copy(feat_hbm.at[0, pl.ds(base, rows_per_w)], cu_v,
                              s1)
        c2 = pltpu.async_copy(feat_hbm.at[1, pl.ds(base, rows_per_w)], co_v,
                              s2)
        c1.wait()
        c2.wait()
        c0.wait()

        lane = lax.iota(jnp.int32, 16)
        is_cust = lane < DIM

        def body(g, carry):
            cuv = cu_v[pl.ds(g * 16, 16)] * DIM
            cov = co_v[pl.ds(g * 16, 16)] * DIM + (VOCAB * DIM - DIM)
            for t in range(16):
                addr = lane + jnp.where(is_cust, cuv[t], cov[t])
                v = plsc.load_gather(tab_v, [addr])
                obuf[2 * g + t // 8, pl.ds((t % 8) * 16, 16)] = v
            return carry

        @plsc.parallel_loop(0, groups)
        def _(g):
            body(g, 0)

        pltpu.sync_copy(obuf, out_hbm.at[pl.ds(wid * prows_w, prows_w), :])

    return gather_kernel


def _mlp_body(p_ref, w1_ref, b1_ref, w2_ref, b2_ref, out_ref):
    f32, bf16 = jnp.float32, jnp.bfloat16
    w1t = jnp.concatenate([jnp.concatenate([w1_ref[...]] * 8, axis=1)] * 8,
                          axis=0)
    r1 = lax.broadcasted_iota(jnp.int32, (128, 8 * HIDDEN), 0) // 16
    c1 = lax.broadcasted_iota(jnp.int32, (128, 8 * HIDDEN), 1) // HIDDEN
    bd1 = jnp.where(r1 == c1, w1t, 0.0).astype(bf16)
    b1t = jnp.concatenate([b1_ref[...]] * 8, axis=1)

    h = jnp.dot(p_ref[...].astype(bf16), bd1, preferred_element_type=f32)
    h = jnp.maximum(h + b1t, 0.0)

    w2t = jnp.concatenate([jnp.concatenate([w2_ref[...]] * 8, axis=0)] * 8,
                          axis=1)
    r2 = lax.broadcasted_iota(jnp.int32, (8 * HIDDEN, 8), 0) // HIDDEN
    c2 = lax.broadcasted_iota(jnp.int32, (8 * HIDDEN, 8), 1)
    bd2 = jnp.where(r2 == c2, w2t, 0.0).astype(bf16)

    z = jnp.dot(h.astype(bf16), bd2, preferred_element_type=f32)
    out_ref[...] = 1.0 / (1.0 + jnp.exp(-(z + b2_ref[...])))


def kernel(features, customers_emb, content_emb, W1, b1, W2, b2):
    batch = features.shape[0]
    prows = batch // 8

    tab = jnp.concatenate([customers_emb, content_emb], axis=0).reshape(-1)
    ft = features.T
    packed = _make_gather(batch)(tab, ft)

    zp = pl.pallas_call(
        _mlp_body,
        grid=(2,),
        in_specs=[
            pl.BlockSpec((prows // 2, 16 * DIM), lambda i: (i, 0)),
            pl.BlockSpec((2 * DIM, HIDDEN), lambda i: (0, 0)),
            pl.BlockSpec((1, HIDDEN), lambda i: (0, 0)),
            pl.BlockSpec((HIDDEN, 1), lambda i: (0, 0)),
            pl.BlockSpec((1, 1), lambda i: (0, 0)),
        ],
        out_specs=pl.BlockSpec((prows // 2, 8), lambda i: (i, 0)),
        out_shape=jax.ShapeDtypeStruct((prows, 8), jnp.float32),
    )(packed, W1, b1.reshape(1, HIDDEN), W2, b2.reshape(1, 1))

    return zp.reshape(batch, 1)
